# baseline (speedup 1.0000x reference)
import functools

import jax
import jax.numpy as jnp
from jax.experimental import pallas as pl

B, N, D, E = 16, 512, 32, 4
TN = 128


def _ggnn_kernel(x_ref, xt_ref, edges_ref, We_ref, Wz_ref, Uz_ref, bz_ref,
                 Wr_ref, Ur_ref, br_ref, Wh_ref, Uh_ref, bh_ref,
                 Wo_ref, bo_ref, out_ref):
    x = x_ref[0]
    xt = xt_ref[0]

    m = jnp.zeros((TN, D), dtype=jnp.float32)
    for e in range(E):
        tx = jnp.dot(x, We_ref[e], preferred_element_type=jnp.float32)
        m = m + jnp.dot(edges_ref[0, e], tx,
                        preferred_element_type=jnp.float32)

    z = jax.nn.sigmoid(jnp.dot(m, Wz_ref[...]) + jnp.dot(xt, Uz_ref[...])
                       + bz_ref[...])
    r = jax.nn.sigmoid(jnp.dot(m, Wr_ref[...]) + jnp.dot(xt, Ur_ref[...])
                       + br_ref[...])
    h_til = jnp.tanh(jnp.dot(m, Wh_ref[...]) + jnp.dot(r * xt, Uh_ref[...])
                     + bh_ref[...])
    h = (1.0 - z) * xt + z * h_til

    logits = jnp.dot(h, Wo_ref[...]) + bo_ref[...]
    lmax = jnp.max(logits, axis=1, keepdims=True)
    shifted = logits - lmax
    lse = jnp.log(jnp.sum(jnp.exp(shifted), axis=1, keepdims=True))
    out_ref[0] = shifted - lse


@functools.partial(jax.jit, static_argnames=())
def kernel(x_padded, x_lengths, edges, fingers, W_edge, Wz, Uz, bz,
           Wr, Ur, br, Wh, Uh, bh, W_out, b_out):
    del x_lengths, fingers
    n_tiles = N // TN
    grid = (B, n_tiles)

    full = lambda b, t: (0, 0)
    out = pl.pallas_call(
        _ggnn_kernel,
        grid=grid,
        in_specs=[
            pl.BlockSpec((1, N, D), lambda b, t: (b, 0, 0)),
            pl.BlockSpec((1, TN, D), lambda b, t: (b, t, 0)),
            pl.BlockSpec((1, E, TN, N), lambda b, t: (b, 0, t, 0)),
            pl.BlockSpec((E, D, D), lambda b, t: (0, 0, 0)),
            pl.BlockSpec((D, D), full),
            pl.BlockSpec((D, D), full),
            pl.BlockSpec((1, D), full),
            pl.BlockSpec((D, D), full),
            pl.BlockSpec((D, D), full),
            pl.BlockSpec((1, D), full),
            pl.BlockSpec((D, D), full),
            pl.BlockSpec((D, D), full),
            pl.BlockSpec((1, D), full),
            pl.BlockSpec((D, 5), full),
            pl.BlockSpec((1, 5), full),
        ],
        out_specs=pl.BlockSpec((1, TN, 5), lambda b, t: (b, t, 0)),
        out_shape=jax.ShapeDtypeStruct((B, N, 5), jnp.float32),
    )(x_padded, x_padded, edges, W_edge, Wz, Uz, bz.reshape(1, D),
      Wr, Ur, br.reshape(1, D), Wh, Uh, bh.reshape(1, D),
      W_out, b_out.reshape(1, 5))
    return out


# speedup vs baseline: 2.0794x; 2.0794x over previous
import jax
import jax.numpy as jnp
from jax.experimental import pallas as pl

B, N, D, E = 16, 512, 32, 4


def _ggnn_kernel(x_ref, edges_ref, We_ref, Wz_ref, Uz_ref, bz_ref,
                 Wr_ref, Ur_ref, br_ref, Wh_ref, Uh_ref, bh_ref,
                 Wo_ref, bo_ref, out_ref):
    x = x_ref[0]

    m = jnp.zeros((N, D), dtype=jnp.float32)
    for e in range(E):
        tx = jnp.dot(x, We_ref[e], preferred_element_type=jnp.float32)
        m = m + jnp.dot(edges_ref[0, e], tx,
                        preferred_element_type=jnp.float32)

    z = jax.nn.sigmoid(jnp.dot(m, Wz_ref[...]) + jnp.dot(x, Uz_ref[...])
                       + bz_ref[...])
    r = jax.nn.sigmoid(jnp.dot(m, Wr_ref[...]) + jnp.dot(x, Ur_ref[...])
                       + br_ref[...])
    h_til = jnp.tanh(jnp.dot(m, Wh_ref[...]) + jnp.dot(r * x, Uh_ref[...])
                     + bh_ref[...])
    h = (1.0 - z) * x + z * h_til

    logits = jnp.dot(h, Wo_ref[...]) + bo_ref[...]
    lmax = jnp.max(logits, axis=1, keepdims=True)
    shifted = logits - lmax
    lse = jnp.log(jnp.sum(jnp.exp(shifted), axis=1, keepdims=True))
    out_ref[0] = shifted - lse


@jax.jit
def kernel(x_padded, x_lengths, edges, fingers, W_edge, Wz, Uz, bz,
           Wr, Ur, br, Wh, Uh, bh, W_out, b_out):
    del x_lengths, fingers
    grid = (B,)

    full = lambda b: (0, 0)
    out = pl.pallas_call(
        _ggnn_kernel,
        grid=grid,
        in_specs=[
            pl.BlockSpec((1, N, D), lambda b: (b, 0, 0)),
            pl.BlockSpec((1, E, N, N), lambda b: (b, 0, 0, 0)),
            pl.BlockSpec((E, D, D), lambda b: (0, 0, 0)),
            pl.BlockSpec((D, D), full),
            pl.BlockSpec((D, D), full),
            pl.BlockSpec((1, D), full),
            pl.BlockSpec((D, D), full),
            pl.BlockSpec((D, D), full),
            pl.BlockSpec((1, D), full),
            pl.BlockSpec((D, D), full),
            pl.BlockSpec((D, D), full),
            pl.BlockSpec((1, D), full),
            pl.BlockSpec((D, 5), full),
            pl.BlockSpec((1, 5), full),
        ],
        out_specs=pl.BlockSpec((1, N, 5), lambda b: (b, 0, 0)),
        out_shape=jax.ShapeDtypeStruct((B, N, 5), jnp.float32),
    )(x_padded, edges, W_edge, Wz, Uz, bz.reshape(1, D),
      Wr, Ur, br.reshape(1, D), Wh, Uh, bh.reshape(1, D),
      W_out, b_out.reshape(1, 5))
    return out


# speedup vs baseline: 2.2177x; 1.0665x over previous
import jax
import jax.numpy as jnp
from jax.experimental import pallas as pl

B, N, D, E = 16, 512, 32, 4


def _ggnn_kernel(x_ref, edges_ref, We_ref, Wz_ref, Uz_ref, bz_ref,
                 Wr_ref, Ur_ref, br_ref, Wh_ref, Uh_ref, bh_ref,
                 Wo_ref, bo_ref, out_ref):
    x = x_ref[0]

    m = jnp.zeros((N, D), dtype=jnp.float32)
    for e in range(E):
        tx = jnp.dot(x, We_ref[e], preferred_element_type=jnp.float32)
        m = m + jnp.dot(edges_ref[0, e].astype(jnp.bfloat16),
                        tx.astype(jnp.bfloat16),
                        preferred_element_type=jnp.float32)

    z = jax.nn.sigmoid(jnp.dot(m, Wz_ref[...]) + jnp.dot(x, Uz_ref[...])
                       + bz_ref[...])
    r = jax.nn.sigmoid(jnp.dot(m, Wr_ref[...]) + jnp.dot(x, Ur_ref[...])
                       + br_ref[...])
    h_til = jnp.tanh(jnp.dot(m, Wh_ref[...]) + jnp.dot(r * x, Uh_ref[...])
                     + bh_ref[...])
    h = (1.0 - z) * x + z * h_til

    logits = jnp.dot(h, Wo_ref[...]) + bo_ref[...]
    lmax = jnp.max(logits, axis=1, keepdims=True)
    shifted = logits - lmax
    lse = jnp.log(jnp.sum(jnp.exp(shifted), axis=1, keepdims=True))
    out_ref[0] = shifted - lse


@jax.jit
def kernel(x_padded, x_lengths, edges, fingers, W_edge, Wz, Uz, bz,
           Wr, Ur, br, Wh, Uh, bh, W_out, b_out):
    del x_lengths, fingers
    grid = (B,)

    full = lambda b: (0, 0)
    out = pl.pallas_call(
        _ggnn_kernel,
        grid=grid,
        in_specs=[
            pl.BlockSpec((1, N, D), lambda b: (b, 0, 0)),
            pl.BlockSpec((1, E, N, N), lambda b: (b, 0, 0, 0)),
            pl.BlockSpec((E, D, D), lambda b: (0, 0, 0)),
            pl.BlockSpec((D, D), full),
            pl.BlockSpec((D, D), full),
            pl.BlockSpec((1, D), full),
            pl.BlockSpec((D, D), full),
            pl.BlockSpec((D, D), full),
            pl.BlockSpec((1, D), full),
            pl.BlockSpec((D, D), full),
            pl.BlockSpec((D, D), full),
            pl.BlockSpec((1, D), full),
            pl.BlockSpec((D, 5), full),
            pl.BlockSpec((1, 5), full),
        ],
        out_specs=pl.BlockSpec((1, N, 5), lambda b: (b, 0, 0)),
        out_shape=jax.ShapeDtypeStruct((B, N, 5), jnp.float32),
    )(x_padded, edges, W_edge, Wz, Uz, bz.reshape(1, D),
      Wr, Ur, br.reshape(1, D), Wh, Uh, bh.reshape(1, D),
      W_out, b_out.reshape(1, 5))
    return out
